# baseline (speedup 1.0000x reference)
import jax
import jax.numpy as jnp
from jax.experimental import pallas as pl
from jax.experimental.pallas import tpu as pltpu

_EPS = 1e-12

B, C, E, D = 4096, 1000, 20, 128
BLK_B = 512
GRID = B // BLK_B


def _fused_kernel(x_ref, ex_ref, dists_ref, preds_ref, means_ref, msq_ref):
    i = pl.program_id(0)

    @pl.when(i == 0)
    def _compute_means():
        ex = ex_ref[...]
        n = jnp.sqrt(jnp.sum(ex * ex, axis=-1, keepdims=True))
        feats = ex / jnp.maximum(n, _EPS)
        mu = jnp.mean(feats, axis=1)
        mn = jnp.sqrt(jnp.sum(mu * mu, axis=-1, keepdims=True))
        means = mu / jnp.maximum(mn, _EPS)
        means_ref[...] = means
        msq_ref[...] = jnp.sum(means * means, axis=-1, keepdims=True)

    xb = x_ref[...]
    xn = jnp.sqrt(jnp.sum(xb * xb, axis=-1, keepdims=True))
    f = xb / jnp.maximum(xn, _EPS)
    x_sq = jnp.sum(f * f, axis=-1, keepdims=True)

    dot = jax.lax.dot_general(
        f, means_ref[...],
        dimension_numbers=(((1,), (1,)), ((), ())),
        preferred_element_type=jnp.float32,
    )
    dists = x_sq - 2.0 * dot + msq_ref[...].reshape(1, C)
    dists_ref[...] = dists
    preds_ref[0, 0, :] = jnp.argmin(dists, axis=-1).astype(jnp.int32)


def kernel(x, exemplar_features):
    dists, preds = pl.pallas_call(
        _fused_kernel,
        grid=(GRID,),
        in_specs=[
            pl.BlockSpec((BLK_B, D), lambda i: (i, 0)),
            pl.BlockSpec((C, E, D), lambda i: (0, 0, 0)),
        ],
        out_specs=[
            pl.BlockSpec((BLK_B, C), lambda i: (i, 0)),
            pl.BlockSpec((1, 1, BLK_B), lambda i: (i, 0, 0)),
        ],
        out_shape=[
            jax.ShapeDtypeStruct((B, C), jnp.float32),
            jax.ShapeDtypeStruct((GRID, 1, BLK_B), jnp.int32),
        ],
        scratch_shapes=[
            pltpu.VMEM((C, D), jnp.float32),
            pltpu.VMEM((C, 1), jnp.float32),
        ],
    )(x, exemplar_features)
    return preds.reshape(B), dists
